# Initial kernel scaffold; baseline (speedup 1.0000x reference)
import jax
import jax.numpy as jnp
from jax.experimental import pallas as pl


def kernel(x, pe, rel_table, alpha):
    raise NotImplementedError("write your pallas kernel here")


# speedup vs baseline: 1.1025x; 1.1025x over previous
import functools

import jax
import jax.numpy as jnp
from jax import lax
from jax.experimental import pallas as pl
from jax.experimental.pallas import tpu as pltpu
from jax.experimental.pallas import tpu_sc as plsc

NC = 2
NS = 16
NW = NC * NS
LANES = 16


def _blend_call(pe_flat, rel_flat, alpha_vec, seq_len, d, rel_row0):
    rows_per_w = seq_len // NW
    ch_rows = min(16, rows_per_w)
    n_ch = rows_per_w // ch_rows
    chunk = ch_rows * d

    mesh = plsc.VectorSubcoreMesh(
        core_axis_name="c", subcore_axis_name="s",
        num_cores=NC, num_subcores=NS)

    @functools.partial(
        pl.kernel,
        out_type=jax.ShapeDtypeStruct((seq_len * d,), jnp.float32),
        mesh=mesh,
        scratch_types=[
            pltpu.VMEM((LANES,), jnp.float32),
            pltpu.VMEM((chunk,), jnp.float32),
            pltpu.VMEM((chunk,), jnp.float32),
            pltpu.VMEM((chunk,), jnp.float32),
        ],
    )
    def run(pe_hbm, rel_hbm, al_hbm, out_hbm, al_v, pe_v, rel_v, out_v):
        wid = lax.axis_index("s") * NC + lax.axis_index("c")
        base_row = wid * rows_per_w
        pltpu.sync_copy(al_hbm, al_v)
        a = al_v[...]
        b = 1.0 - a

        def chunk_body(c, _):
            row = base_row + c * ch_rows
            pltpu.sync_copy(pe_hbm.at[pl.ds(row * d, chunk)], pe_v)
            pltpu.sync_copy(rel_hbm.at[pl.ds((rel_row0 + row) * d, chunk)],
                            rel_v)

            def vec_body(i, _):
                sl = pl.ds(i * LANES, LANES)
                out_v[sl] = a * pe_v[sl] + b * rel_v[sl]
                return ()

            lax.fori_loop(0, chunk // LANES, vec_body, ())
            pltpu.sync_copy(out_v, out_hbm.at[pl.ds(row * d, chunk)])
            return ()

        lax.fori_loop(0, n_ch, chunk_body, ())

    return run(pe_flat, rel_flat, alpha_vec)


def kernel(x, pe, rel_table, alpha):
    seq_len = x.shape[1]
    d = pe.shape[-1]
    rel_row0 = pe.shape[1] - seq_len

    pe_flat = pe.reshape(-1)
    rel_flat = rel_table.reshape(-1)
    alpha_vec = jnp.full((LANES,), alpha, dtype=jnp.float32)

    out = _blend_call(pe_flat, rel_flat, alpha_vec, seq_len, d, rel_row0)
    return out.reshape(1, seq_len, d)


# speedup vs baseline: 1.4021x; 1.2717x over previous
import functools

import jax
import jax.numpy as jnp
from jax import lax
from jax.experimental import pallas as pl
from jax.experimental.pallas import tpu as pltpu
from jax.experimental.pallas import tpu_sc as plsc

NC = 2
NS = 16
NW = NC * NS
LANES = 16


def _blend_call(pe_flat, rel_flat, alpha_vec, seq_len, d, rel_row0):
    rows_per_w = seq_len // NW
    ch_rows = min(16, rows_per_w)
    n_ch = rows_per_w // ch_rows
    chunk = ch_rows * d

    mesh = plsc.VectorSubcoreMesh(
        core_axis_name="c", subcore_axis_name="s",
        num_cores=NC, num_subcores=NS)

    @functools.partial(
        pl.kernel,
        out_type=jax.ShapeDtypeStruct((seq_len * d,), jnp.float32),
        mesh=mesh,
        scratch_types=[
            pltpu.VMEM((LANES,), jnp.float32),
            pltpu.VMEM((chunk,), jnp.float32),
            pltpu.VMEM((chunk,), jnp.float32),
            pltpu.VMEM((chunk,), jnp.float32),
            pltpu.VMEM((chunk,), jnp.float32),
            pltpu.VMEM((chunk,), jnp.float32),
            pltpu.VMEM((chunk,), jnp.float32),
            pltpu.SemaphoreType.DMA,
            pltpu.SemaphoreType.DMA,
            pltpu.SemaphoreType.DMA,
            pltpu.SemaphoreType.DMA,
        ],
    )
    def run(pe_hbm, rel_hbm, al_hbm, out_hbm, al_v,
            pe_b0, pe_b1, rel_b0, rel_b1, out_b0, out_b1,
            si0, si1, so0, so1):
        wid = lax.axis_index("s") * NC + lax.axis_index("c")
        base_row = wid * rows_per_w
        pltpu.sync_copy(al_hbm, al_v)
        a = al_v[...]
        b = 1.0 - a

        pe_bufs = [pe_b0, pe_b1]
        rel_bufs = [rel_b0, rel_b1]
        out_bufs = [out_b0, out_b1]
        s_in = [si0, si1]
        s_out = [so0, so1]

        def in_srcs(c):
            row = base_row + c * ch_rows
            return (pe_hbm.at[pl.ds(row * d, chunk)],
                    rel_hbm.at[pl.ds((rel_row0 + row) * d, chunk)])

        def out_dst(c):
            row = base_row + c * ch_rows
            return out_hbm.at[pl.ds(row * d, chunk)]

        ps, rs = in_srcs(0)
        pltpu.async_copy(ps, pe_bufs[0], s_in[0])
        pltpu.async_copy(rs, rel_bufs[0], s_in[0])

        for c in range(n_ch):
            s = c % 2
            if c + 1 < n_ch:
                ns = (c + 1) % 2
                ps, rs = in_srcs(c + 1)
                pltpu.async_copy(ps, pe_bufs[ns], s_in[ns])
                pltpu.async_copy(rs, rel_bufs[ns], s_in[ns])
            psc, rsc = in_srcs(c)
            pltpu.make_async_copy(psc, pe_bufs[s], s_in[s]).wait()
            pltpu.make_async_copy(rsc, rel_bufs[s], s_in[s]).wait()
            if c >= 2:
                pltpu.make_async_copy(out_bufs[s], out_dst(c - 2),
                                      s_out[s]).wait()
            pv, rv, ov = pe_bufs[s], rel_bufs[s], out_bufs[s]

            @plsc.parallel_loop(0, chunk // LANES, 1, unroll=8)
            def _(i):
                sl = pl.ds(i * LANES, LANES)
                ov[sl] = a * pv[sl] + b * rv[sl]

            pltpu.async_copy(ov, out_dst(c), s_out[s])

        for c in range(max(0, n_ch - 2), n_ch):
            s = c % 2
            pltpu.make_async_copy(out_bufs[s], out_dst(c), s_out[s]).wait()

    return run(pe_flat, rel_flat, alpha_vec)


def kernel(x, pe, rel_table, alpha):
    seq_len = x.shape[1]
    d = pe.shape[-1]
    rel_row0 = pe.shape[1] - seq_len

    pe_flat = pe.reshape(-1)
    rel_flat = rel_table.reshape(-1)
    alpha_vec = jnp.full((LANES,), alpha, dtype=jnp.float32)

    out = _blend_call(pe_flat, rel_flat, alpha_vec, seq_len, d, rel_row0)
    return out.reshape(1, seq_len, d)


# speedup vs baseline: 4.7146x; 3.3626x over previous
import functools

import jax
import jax.numpy as jnp
from jax import lax
from jax.experimental import pallas as pl
from jax.experimental.pallas import tpu as pltpu
from jax.experimental.pallas import tpu_sc as plsc

NC = 2
NS = 16
NW = NC * NS
LANES = 16


def _blend_call(pe2, rel2, alpha_vec, seq_len, d, rel_row0):
    rows_per_w = seq_len // NW
    ch_rows = min(16, rows_per_w)
    n_ch = rows_per_w // ch_rows

    mesh = plsc.VectorSubcoreMesh(
        core_axis_name="c", subcore_axis_name="s",
        num_cores=NC, num_subcores=NS)

    @functools.partial(
        pl.kernel,
        out_type=jax.ShapeDtypeStruct((seq_len, d), jnp.float32),
        mesh=mesh,
        compiler_params=pltpu.CompilerParams(use_tc_tiling_on_sc=True),
        scratch_types=[
            pltpu.VMEM((LANES,), jnp.float32),
            pltpu.VMEM((ch_rows, d), jnp.float32),
            pltpu.VMEM((ch_rows, d), jnp.float32),
            pltpu.VMEM((ch_rows, d), jnp.float32),
            pltpu.VMEM((ch_rows, d), jnp.float32),
            pltpu.VMEM((ch_rows, d), jnp.float32),
            pltpu.VMEM((ch_rows, d), jnp.float32),
            pltpu.SemaphoreType.DMA,
            pltpu.SemaphoreType.DMA,
            pltpu.SemaphoreType.DMA,
            pltpu.SemaphoreType.DMA,
        ],
    )
    def run(pe_hbm, rel_hbm, al_hbm, out_hbm, al_v,
            pe_b0, pe_b1, rel_b0, rel_b1, out_b0, out_b1,
            si0, si1, so0, so1):
        wid = lax.axis_index("s") * NC + lax.axis_index("c")
        base_row = wid * rows_per_w
        pltpu.sync_copy(al_hbm, al_v)
        a = al_v[...]
        b = 1.0 - a

        pe_bufs = [pe_b0, pe_b1]
        rel_bufs = [rel_b0, rel_b1]
        out_bufs = [out_b0, out_b1]
        s_in = [si0, si1]
        s_out = [so0, so1]

        def in_srcs(c):
            row = base_row + c * ch_rows
            return (pe_hbm.at[pl.ds(row, ch_rows), :],
                    rel_hbm.at[pl.ds(rel_row0 + row, ch_rows), :])

        def out_dst(c):
            row = base_row + c * ch_rows
            return out_hbm.at[pl.ds(row, ch_rows), :]

        ps, rs = in_srcs(0)
        pltpu.async_copy(ps, pe_bufs[0], s_in[0])
        pltpu.async_copy(rs, rel_bufs[0], s_in[0])

        for c in range(n_ch):
            s = c % 2
            if c + 1 < n_ch:
                ns = (c + 1) % 2
                ps, rs = in_srcs(c + 1)
                pltpu.async_copy(ps, pe_bufs[ns], s_in[ns])
                pltpu.async_copy(rs, rel_bufs[ns], s_in[ns])
            psc, rsc = in_srcs(c)
            pltpu.make_async_copy(psc, pe_bufs[s], s_in[s]).wait()
            pltpu.make_async_copy(rsc, rel_bufs[s], s_in[s]).wait()
            if c >= 2:
                pltpu.make_async_copy(out_bufs[s], out_dst(c - 2),
                                      s_out[s]).wait()
            pv, rv, ov = pe_bufs[s], rel_bufs[s], out_bufs[s]
            n_vec = d // LANES

            @plsc.parallel_loop(0, ch_rows * n_vec, 1, unroll=8)
            def _(i):
                r = i // n_vec
                sl = pl.ds((i % n_vec) * LANES, LANES)
                ov[r, sl] = a * pv[r, sl] + b * rv[r, sl]

            pltpu.async_copy(ov, out_dst(c), s_out[s])

        for c in range(max(0, n_ch - 2), n_ch):
            s = c % 2
            pltpu.make_async_copy(out_bufs[s], out_dst(c), s_out[s]).wait()

    return run(pe2, rel2, alpha_vec)


def kernel(x, pe, rel_table, alpha):
    seq_len = x.shape[1]
    d = pe.shape[-1]
    rel_row0 = pe.shape[1] - seq_len

    pe2 = pe.reshape(pe.shape[1], d)
    alpha_vec = jnp.full((LANES,), alpha, dtype=jnp.float32)

    out = _blend_call(pe2, rel_table, alpha_vec, seq_len, d, rel_row0)
    return out.reshape(1, seq_len, d)


# speedup vs baseline: 4.7744x; 1.0127x over previous
import functools

import jax
import jax.numpy as jnp
from jax import lax
from jax.experimental import pallas as pl
from jax.experimental.pallas import tpu as pltpu
from jax.experimental.pallas import tpu_sc as plsc

NC = 2
NS = 16
NW = NC * NS
LANES = 16


N_IN = 4
N_OUT = 2


def _blend_call(pe2, rel2, alpha_vec, seq_len, d, rel_row0):
    rows_per_w = seq_len // NW
    ch_rows = min(8, rows_per_w)
    n_ch = rows_per_w // ch_rows

    mesh = plsc.VectorSubcoreMesh(
        core_axis_name="c", subcore_axis_name="s",
        num_cores=NC, num_subcores=NS)

    buf = pltpu.VMEM((ch_rows, d), jnp.float32)
    sem = pltpu.SemaphoreType.DMA

    @functools.partial(
        pl.kernel,
        out_type=jax.ShapeDtypeStruct((seq_len, d), jnp.float32),
        mesh=mesh,
        compiler_params=pltpu.CompilerParams(use_tc_tiling_on_sc=True),
        scratch_types=(
            [pltpu.VMEM((LANES,), jnp.float32)]
            + [buf] * (2 * N_IN + N_OUT)
            + [sem] * (N_IN + N_OUT)
        ),
    )
    def run(pe_hbm, rel_hbm, al_hbm, out_hbm, al_v, *bufs_and_sems):
        pe_bufs = list(bufs_and_sems[0:N_IN])
        rel_bufs = list(bufs_and_sems[N_IN:2 * N_IN])
        out_bufs = list(bufs_and_sems[2 * N_IN:2 * N_IN + N_OUT])
        s_in = list(bufs_and_sems[2 * N_IN + N_OUT:2 * N_IN + N_OUT + N_IN])
        s_out = list(bufs_and_sems[2 * N_IN + N_OUT + N_IN:])

        wid = lax.axis_index("s") * NC + lax.axis_index("c")
        base_row = wid * rows_per_w
        pltpu.sync_copy(al_hbm, al_v)
        a = al_v[...]
        b = 1.0 - a
        n_vec = d // LANES

        def in_srcs(c):
            row = base_row + c * ch_rows
            return (pe_hbm.at[pl.ds(row, ch_rows), :],
                    rel_hbm.at[pl.ds(rel_row0 + row, ch_rows), :])

        def out_dst(c):
            row = base_row + c * ch_rows
            return out_hbm.at[pl.ds(row, ch_rows), :]

        def start_in(c):
            s = c % N_IN
            ps, rs = in_srcs(c)
            pltpu.async_copy(ps, pe_bufs[s], s_in[s])
            pltpu.async_copy(rs, rel_bufs[s], s_in[s])

        for c in range(min(N_IN - 1, n_ch)):
            start_in(c)

        for c in range(n_ch):
            s = c % N_IN
            if c + N_IN - 1 < n_ch:
                start_in(c + N_IN - 1)
            psc, rsc = in_srcs(c)
            pltpu.make_async_copy(psc, pe_bufs[s], s_in[s]).wait()
            pltpu.make_async_copy(rsc, rel_bufs[s], s_in[s]).wait()
            so = c % N_OUT
            if c >= N_OUT:
                pltpu.make_async_copy(out_bufs[so], out_dst(c - N_OUT),
                                      s_out[so]).wait()
            pv, rv, ov = pe_bufs[s], rel_bufs[s], out_bufs[so]

            @plsc.parallel_loop(0, ch_rows * n_vec, 1, unroll=8)
            def _(i):
                r = i // n_vec
                sl = pl.ds((i % n_vec) * LANES, LANES)
                ov[r, sl] = a * pv[r, sl] + b * rv[r, sl]

            pltpu.async_copy(ov, out_dst(c), s_out[so])

        for c in range(max(0, n_ch - N_OUT), n_ch):
            so = c % N_OUT
            pltpu.make_async_copy(out_bufs[so], out_dst(c), s_out[so]).wait()

    return run(pe2, rel2, alpha_vec)


def kernel(x, pe, rel_table, alpha):
    seq_len = x.shape[1]
    d = pe.shape[-1]
    rel_row0 = pe.shape[1] - seq_len

    pe2 = pe.reshape(pe.shape[1], d)
    alpha_vec = jnp.full((LANES,), alpha, dtype=jnp.float32)

    out = _blend_call(pe2, rel_table, alpha_vec, seq_len, d, rel_row0)
    return out.reshape(1, seq_len, d)
